# Initial kernel scaffold; baseline (speedup 1.0000x reference)
import jax
import jax.numpy as jnp
from jax.experimental import pallas as pl


def kernel(x, knn_matrix, w1, conv_w, conv_b):
    raise NotImplementedError("write your pallas kernel here")


# speedup vs baseline: 4.7585x; 4.7585x over previous
import functools

import jax
import jax.numpy as jnp
from jax import lax
from jax.experimental import pallas as pl
from jax.experimental.pallas import tpu as pltpu
from jax.experimental.pallas import tpu_sc as plsc

B, N, D, C, K = 8, 256, 25, 32, 20
DC = D * C
ROWS = B * N
LANES = 16
NC, NS = 2, 16
NW = NC * NS
RPW = ROWS // NW
G = 4
NCH = RPW // G
IPC = G * K
NCOL = DC // LANES


def _fm_body(x_hbm, idx_hbm, w1t_hbm, out_hbm, idx_v, w1t_v, w_v, rows_v,
             out_v, sem):
    wid = lax.axis_index("s") * NC + lax.axis_index("c")

    pltpu.sync_copy(idx_hbm.at[wid], idx_v)
    pltpu.sync_copy(w1t_hbm, w1t_v)

    s_lo = jnp.zeros((LANES,), jnp.float32)
    s_hi = jnp.zeros((LANES,), jnp.float32)
    for k in range(K):
        a = w1t_v[k, 0]
        b = w1t_v[k, 1]
        s_lo = s_lo + a * a
        s_hi = s_hi + b * b
    inv_lo = 1.0 / s_lo
    inv_hi = 1.0 / s_hi
    for k in range(K):
        a = w1t_v[k, 0]
        b = w1t_v[k, 1]
        wn_lo = a * a * inv_lo
        wn_hi = b * b * inv_hi

        def put(d, carry):
            lo, hi = carry
            w_v[k, 2 * d] = lo
            w_v[k, 2 * d + 1] = hi
            return carry

        lax.fori_loop(0, D, put, (wn_lo, wn_hi))

    def chunk(j, carry):
        pltpu.async_copy(x_hbm.at[idx_v.at[j]], rows_v, sem).wait()

        def col(i, c2):
            sl = pl.ds(i * LANES, LANES)
            accs = [jnp.zeros((LANES,), jnp.float32) for _ in range(G)]
            for k in range(K):
                wk = w_v[k, i]
                for g in range(G):
                    accs[g] = accs[g] + rows_v[g * K + k, sl] * wk
            for g in range(G):
                out_v[g, sl] = accs[g]
            return c2

        lax.fori_loop(0, NCOL, col, 0)
        pltpu.sync_copy(out_v, out_hbm.at[pl.ds(wid * RPW + j * G, G)])
        return carry

    lax.fori_loop(0, NCH, chunk, 0)


@functools.partial(jax.jit, static_argnames=())
def _fm_call(x_flat, idx, w1t):
    mesh = plsc.VectorSubcoreMesh(core_axis_name="c", subcore_axis_name="s")
    run = functools.partial(
        pl.kernel,
        mesh=mesh,
        out_type=jax.ShapeDtypeStruct((ROWS, DC), jnp.float32),
        scratch_types=[
            pltpu.VMEM((NCH, IPC), jnp.int32),
            pltpu.VMEM((K, 2, LANES), jnp.float32),
            pltpu.VMEM((K, NCOL, LANES), jnp.float32),
            pltpu.VMEM((IPC, DC), jnp.float32),
            pltpu.VMEM((G, DC), jnp.float32),
            pltpu.SemaphoreType.DMA,
        ],
        compiler_params=pltpu.CompilerParams(use_tc_tiling_on_sc=False),
    )(_fm_body)
    return run(x_flat, idx, w1t)


def kernel(x, knn_matrix, w1, conv_w, conv_b):
    del conv_w, conv_b
    x_flat = x.reshape(ROWS, DC)
    flat_idx = (knn_matrix.astype(jnp.int32)
                + (jnp.arange(B, dtype=jnp.int32) * N).reshape(B, 1, 1))
    idx = flat_idx.reshape(NW, NCH, IPC)
    w1t = w1.T.reshape(K, 2, LANES)
    out = _fm_call(x_flat, idx, w1t)
    return out.reshape(B, N, D, C)


# speedup vs baseline: 6.2292x; 1.3091x over previous
import functools

import jax
import jax.numpy as jnp
from jax import lax
from jax.experimental import pallas as pl
from jax.experimental.pallas import tpu as pltpu
from jax.experimental.pallas import tpu_sc as plsc

B, N, D, C, K = 8, 256, 25, 32, 20
DC = D * C
ROWS = B * N
LANES = 16
NC, NS = 2, 16
NW = NC * NS
RPW = ROWS // NW
G = 2
NCH = RPW // G
IPC = G * K
NB = 2


def _fm_body(x_hbm, idx_hbm, w1t_hbm, out_hbm, idx_v, w1t_v, rows_v, out_v,
             gsems, osems):
    wid = lax.axis_index("s") * NC + lax.axis_index("c")

    pltpu.sync_copy(idx_hbm.at[wid], idx_v)
    pltpu.sync_copy(w1t_hbm, w1t_v)

    sums = []
    for h in range(2):
        s = jnp.zeros((LANES,), jnp.float32)
        for k in range(K):
            a = w1t_v[k, h]
            s = s + a * a
        sums.append(1.0 / s)
    wn = [[w1t_v[k, h] * w1t_v[k, h] * sums[h] for k in range(K)]
          for h in range(2)]

    def start_gather(j, b):
        pltpu.async_copy(x_hbm.at[idx_v.at[j]], rows_v.at[b], gsems.at[b])

    def wait_gather(b):
        pltpu.make_async_copy(x_hbm.at[idx_v.at[0]], rows_v.at[b],
                              gsems.at[b]).wait()

    def wait_out(b):
        pltpu.make_async_copy(out_v.at[b], out_hbm.at[pl.ds(0, G)],
                              osems.at[b]).wait()

    for b in range(NB):
        start_gather(b, b)

    def chunk_pair(j2, carry):
        for bb in range(NB):
            j = j2 * NB + bb
            wait_gather(bb)

            @pl.when(j2 > 0)
            def _():
                wait_out(bb)

            def col(d, c2):
                for h in range(2):
                    sl = pl.ds((2 * d + h) * LANES, LANES)
                    for g in range(G):
                        acc0 = rows_v[bb, g * K, sl] * wn[h][0]
                        acc1 = rows_v[bb, g * K + 1, sl] * wn[h][1]
                        for k in range(2, K, 2):
                            acc0 = acc0 + rows_v[bb, g * K + k, sl] * wn[h][k]
                            acc1 = (acc1
                                    + rows_v[bb, g * K + k + 1, sl]
                                    * wn[h][k + 1])
                        out_v[bb, g, sl] = acc0 + acc1
                return c2

            lax.fori_loop(0, D, col, 0)
            pltpu.async_copy(out_v.at[bb],
                             out_hbm.at[pl.ds(wid * RPW + j * G, G)],
                             osems.at[bb])

            @pl.when(j + NB < NCH)
            def _():
                start_gather(j + NB, bb)
        return carry

    lax.fori_loop(0, NCH // NB, chunk_pair, 0)
    for b in range(NB):
        wait_out(b)


@jax.jit
def _fm_call(x_flat, idx, w1t):
    mesh = plsc.VectorSubcoreMesh(core_axis_name="c", subcore_axis_name="s")
    run = functools.partial(
        pl.kernel,
        mesh=mesh,
        out_type=jax.ShapeDtypeStruct((ROWS, DC), jnp.float32),
        scratch_types=[
            pltpu.VMEM((NCH, IPC), jnp.int32),
            pltpu.VMEM((K, 2, LANES), jnp.float32),
            pltpu.VMEM((NB, IPC, DC), jnp.float32),
            pltpu.VMEM((NB, G, DC), jnp.float32),
            pltpu.SemaphoreType.DMA((NB,)),
            pltpu.SemaphoreType.DMA((NB,)),
        ],
        compiler_params=pltpu.CompilerParams(use_tc_tiling_on_sc=False),
    )(_fm_body)
    return run(x_flat, idx, w1t)


def kernel(x, knn_matrix, w1, conv_w, conv_b):
    del conv_w, conv_b
    x_flat = x.reshape(ROWS, DC)
    flat_idx = (knn_matrix.astype(jnp.int32)
                + (jnp.arange(B, dtype=jnp.int32) * N).reshape(B, 1, 1))
    idx = flat_idx.reshape(NW, NCH, IPC)
    w1t = w1.T.reshape(K, 2, LANES)
    out = _fm_call(x_flat, idx, w1t)
    return out.reshape(B, N, D, C)
